# Initial kernel scaffold; baseline (speedup 1.0000x reference)
import jax
import jax.numpy as jnp
from jax.experimental import pallas as pl


def kernel(ui_uKey, ui_iKey, uEmbeds, iEmbeds, ui_uHyper, ui_iHyper, rows, cols, edgeids):
    raise NotImplementedError("write your pallas kernel here")


# speedup vs baseline: 6.6171x; 6.6171x over previous
import functools

import jax
import jax.numpy as jnp
from jax import lax
from jax.experimental import pallas as pl
from jax.experimental.pallas import tpu as pltpu
from jax.experimental.pallas import tpu_sc as plsc

N_USERS = 50000
N_ITEMS = 50000
LATDIM = 64
E_TOTAL = 1600000

NC = 2
NS = 16
NW = NC * NS

ROWS_BLK = 1000

C = 80
EW = E_TOTAL // NW
NIT = EW // C


def _tables_body(uKey_r, uEmb_r, iKey_r, iEmb_r, uH_r, iH_r, U_r, I_r):
    dn = (((1,), (1,)), ((), ()))
    M = lax.dot_general(uH_r[...], iH_r[...], dn,
                        precision=lax.Precision.HIGHEST,
                        preferred_element_type=jnp.float32)
    uProj = lax.dot_general(uKey_r[...], M, (((1,), (0,)), ((), ())),
                            precision=lax.Precision.HIGHEST,
                            preferred_element_type=jnp.float32)
    U_r[...] = jnp.concatenate([uProj, uEmb_r[...]], axis=1)
    I_r[...] = jnp.concatenate([iKey_r[...], iEmb_r[...]], axis=1)


def _build_tables(uKey, uEmbeds, iKey, iEmbeds, uHyper, iHyper):
    grid = (N_USERS // ROWS_BLK,)
    blk = lambda i: (i, 0)
    full = lambda i: (0, 0)
    return pl.pallas_call(
        _tables_body,
        grid=grid,
        in_specs=[
            pl.BlockSpec((ROWS_BLK, LATDIM), blk),
            pl.BlockSpec((ROWS_BLK, LATDIM), blk),
            pl.BlockSpec((ROWS_BLK, LATDIM), blk),
            pl.BlockSpec((ROWS_BLK, LATDIM), blk),
            pl.BlockSpec((LATDIM, 128), full),
            pl.BlockSpec((LATDIM, 128), full),
        ],
        out_specs=[
            pl.BlockSpec((ROWS_BLK, 2 * LATDIM), blk),
            pl.BlockSpec((ROWS_BLK, 2 * LATDIM), blk),
        ],
        out_shape=[
            jax.ShapeDtypeStruct((N_USERS, 2 * LATDIM), jnp.float32),
            jax.ShapeDtypeStruct((N_ITEMS, 2 * LATDIM), jnp.float32),
        ],
    )(uKey, uEmbeds, iKey, iEmbeds, uHyper, iHyper)


def _edge_body(eid_hbm, rows_hbm, cols_hbm, U_hbm, I_hbm, out_hbm,
               eid_v, usr_v, itm_v, Urows_v, Irows_v, out_v, sem):
    wid = lax.axis_index("s") * NC + lax.axis_index("c")
    iota16 = lax.iota(jnp.int32, 16)

    def it_body(it, carry):
        base = wid * EW + it * C
        pltpu.sync_copy(eid_hbm.at[pl.ds(base, C)], eid_v)
        cp1 = pltpu.async_copy(rows_hbm.at[eid_v], usr_v, sem)
        cp2 = pltpu.async_copy(cols_hbm.at[eid_v], itm_v, sem)
        cp1.wait()
        cp2.wait()
        cp3 = pltpu.async_copy(U_hbm.at[usr_v], Urows_v, sem)
        cp4 = pltpu.async_copy(I_hbm.at[itm_v], Irows_v, sem)
        cp3.wait()
        cp4.wait()

        def grp_body(g, carry2):
            def e_body(e2, vv):
                v1, v2 = vv
                e = g * 16 + e2
                acc1 = jnp.zeros((16,), jnp.float32)
                acc2 = jnp.zeros((16,), jnp.float32)
                for j in range(4):
                    acc1 = acc1 + (Urows_v[e, pl.ds(16 * j, 16)]
                                   * Irows_v[e, pl.ds(16 * j, 16)])
                    acc2 = acc2 + (Urows_v[e, pl.ds(64 + 16 * j, 16)]
                                   * Irows_v[e, pl.ds(64 + 16 * j, 16)])
                lane = iota16 == e2
                v1 = jnp.where(lane, jnp.sum(acc1), v1)
                v2 = jnp.where(lane, jnp.sum(acc2), v2)
                return (v1, v2)

            zeros = jnp.zeros((16,), jnp.float32)
            a, b = lax.fori_loop(0, 16, e_body, (zeros, zeros))
            out_v[pl.ds(g * 16, 16)] = jnp.abs(1.0 / (1.0 + jnp.exp(-a)) - b)
            return carry2

        lax.fori_loop(0, C // 16, grp_body, 0)
        pltpu.sync_copy(out_v, out_hbm.at[pl.ds(base, C)])
        return carry

    lax.fori_loop(0, NIT, it_body, 0)


_edge_kernel = pl.kernel(
    _edge_body,
    out_type=jax.ShapeDtypeStruct((E_TOTAL,), jnp.float32),
    mesh=plsc.VectorSubcoreMesh(core_axis_name="c", subcore_axis_name="s",
                                num_cores=NC, num_subcores=NS),
    compiler_params=pltpu.CompilerParams(needs_layout_passes=False),
    scratch_types=[
        pltpu.VMEM((C,), jnp.int32),
        pltpu.VMEM((C,), jnp.int32),
        pltpu.VMEM((C,), jnp.int32),
        pltpu.VMEM((C, 2 * LATDIM), jnp.float32),
        pltpu.VMEM((C, 2 * LATDIM), jnp.float32),
        pltpu.VMEM((C,), jnp.float32),
        pltpu.SemaphoreType.DMA,
    ],
)


def kernel(ui_uKey, ui_iKey, uEmbeds, iEmbeds, ui_uHyper, ui_iHyper,
           rows, cols, edgeids):
    latdim = ui_uKey.shape[0] * ui_uKey.shape[2]
    uKey = jnp.transpose(ui_uKey, (1, 0, 2)).reshape(-1, latdim)
    iKey = jnp.transpose(ui_iKey, (1, 0, 2)).reshape(-1, latdim)
    U, I = _build_tables(uKey, uEmbeds, iKey, iEmbeds, ui_uHyper, ui_iHyper)
    return _edge_kernel(edgeids, rows, cols, U, I)


# speedup vs baseline: 13.8598x; 2.0945x over previous
import functools

import jax
import jax.numpy as jnp
from jax import lax
from jax.experimental import pallas as pl
from jax.experimental.pallas import tpu as pltpu
from jax.experimental.pallas import tpu_sc as plsc

N_USERS = 50000
N_ITEMS = 50000
LATDIM = 64
E_TOTAL = 1600000

NC = 2
NS = 16
NW = NC * NS

ROWS_BLK = 1000

C = 80
EW = E_TOTAL // NW
NIT = EW // C


def _tables_body(uKey_r, uEmb_r, iKey_r, iEmb_r, uH_r, iH_r, U_r, I_r):
    dn = (((1,), (1,)), ((), ()))
    M = lax.dot_general(uH_r[...], iH_r[...], dn,
                        precision=lax.Precision.HIGHEST,
                        preferred_element_type=jnp.float32)
    uProj = lax.dot_general(uKey_r[...], M, (((1,), (0,)), ((), ())),
                            precision=lax.Precision.HIGHEST,
                            preferred_element_type=jnp.float32)
    U_r[...] = jnp.concatenate([uProj, uEmb_r[...]], axis=1)
    I_r[...] = jnp.concatenate([iKey_r[...], iEmb_r[...]], axis=1)


def _build_tables(uKey, uEmbeds, iKey, iEmbeds, uHyper, iHyper):
    grid = (N_USERS // ROWS_BLK,)
    blk = lambda i: (i, 0)
    full = lambda i: (0, 0)
    return pl.pallas_call(
        _tables_body,
        grid=grid,
        in_specs=[
            pl.BlockSpec((ROWS_BLK, LATDIM), blk),
            pl.BlockSpec((ROWS_BLK, LATDIM), blk),
            pl.BlockSpec((ROWS_BLK, LATDIM), blk),
            pl.BlockSpec((ROWS_BLK, LATDIM), blk),
            pl.BlockSpec((LATDIM, 128), full),
            pl.BlockSpec((LATDIM, 128), full),
        ],
        out_specs=[
            pl.BlockSpec((ROWS_BLK, 2 * LATDIM), blk),
            pl.BlockSpec((ROWS_BLK, 2 * LATDIM), blk),
        ],
        out_shape=[
            jax.ShapeDtypeStruct((N_USERS, 2 * LATDIM), jnp.float32),
            jax.ShapeDtypeStruct((N_ITEMS, 2 * LATDIM), jnp.float32),
        ],
    )(uKey, uEmbeds, iKey, iEmbeds, uHyper, iHyper)


def _edge_body(eid_hbm, rows_hbm, cols_hbm, U_hbm, I_hbm, out_hbm,
               eid_all,
               usr0, usr1, itm0, itm1,
               Ur0, Ur1, Ir0, Ir1, out0, out1,
               semb0, semb1, semc0, semc1, semo0, semo1):
    wid = lax.axis_index("s") * NC + lax.axis_index("c")
    iota16 = lax.iota(jnp.int32, 16)
    usr = (usr0, usr1)
    itm = (itm0, itm1)
    Ur = (Ur0, Ur1)
    Ir = (Ir0, Ir1)
    outb = (out0, out1)
    semb = (semb0, semb1)
    semc = (semc0, semc1)
    semo = (semo0, semo1)
    wbase = wid * EW

    def issue_b(n, p):
        idx = eid_all.at[pl.ds(n * C, C)]
        pltpu.async_copy(rows_hbm.at[idx], usr[p], semb[p])
        pltpu.async_copy(cols_hbm.at[idx], itm[p], semb[p])

    def wait_b(p):
        idx = eid_all.at[pl.ds(0, C)]
        pltpu.make_async_copy(rows_hbm.at[idx], usr[p], semb[p]).wait()
        pltpu.make_async_copy(cols_hbm.at[idx], itm[p], semb[p]).wait()

    def issue_c(p):
        pltpu.async_copy(U_hbm.at[usr[p]], Ur[p], semc[p])
        pltpu.async_copy(I_hbm.at[itm[p]], Ir[p], semc[p])

    def wait_c(p):
        pltpu.make_async_copy(U_hbm.at[usr[p]], Ur[p], semc[p]).wait()
        pltpu.make_async_copy(I_hbm.at[itm[p]], Ir[p], semc[p]).wait()

    def compute(n, p):
        Urows_v, Irows_v, out_v = Ur[p], Ir[p], outb[p]

        def grp_body(g, carry2):
            def e_body(e2, vv):
                v1, v2 = vv
                e = g * 16 + e2
                acc1 = jnp.zeros((16,), jnp.float32)
                acc2 = jnp.zeros((16,), jnp.float32)
                for j in range(4):
                    acc1 = acc1 + (Urows_v[e, pl.ds(16 * j, 16)]
                                   * Irows_v[e, pl.ds(16 * j, 16)])
                    acc2 = acc2 + (Urows_v[e, pl.ds(64 + 16 * j, 16)]
                                   * Irows_v[e, pl.ds(64 + 16 * j, 16)])
                lane = iota16 == e2
                v1 = jnp.where(lane, jnp.sum(acc1), v1)
                v2 = jnp.where(lane, jnp.sum(acc2), v2)
                return (v1, v2)

            zeros = jnp.zeros((16,), jnp.float32)
            a, b = lax.fori_loop(0, 16, e_body, (zeros, zeros))
            out_v[pl.ds(g * 16, 16)] = jnp.abs(1.0 / (1.0 + jnp.exp(-a)) - b)
            return carry2

        lax.fori_loop(0, C // 16, grp_body, 0)
        pltpu.async_copy(out_v, out_hbm.at[pl.ds(wbase + n * C, C)], semo[p])

    def wait_o(p):
        pltpu.make_async_copy(outb[p], out_hbm.at[pl.ds(0, C)], semo[p]).wait()

    pltpu.sync_copy(eid_hbm.at[pl.ds(wbase, EW)], eid_all)
    issue_b(0, 0)
    issue_b(1, 1)
    wait_b(0)
    issue_c(0)

    def half_body(n, p):
        wait_c(p)

        @pl.when(n + 2 < NIT)
        def _():
            issue_b(n + 2, p)

        @pl.when(n + 1 < NIT)
        def _():
            wait_b(1 - p)
            issue_c(1 - p)

        @pl.when(n >= 2)
        def _():
            wait_o(p)

        compute(n, p)

    def pair_body(n2, carry):
        n = 2 * n2
        half_body(n, 0)

        @pl.when(n + 1 < NIT)
        def _():
            half_body(n + 1, 1)

        return carry

    lax.fori_loop(0, (NIT + 1) // 2, pair_body, 0)
    wait_o(0)
    wait_o(1)


_edge_kernel = pl.kernel(
    _edge_body,
    out_type=jax.ShapeDtypeStruct((E_TOTAL,), jnp.float32),
    mesh=plsc.VectorSubcoreMesh(core_axis_name="c", subcore_axis_name="s",
                                num_cores=NC, num_subcores=NS),
    compiler_params=pltpu.CompilerParams(needs_layout_passes=False),
    scratch_types=(
        [pltpu.VMEM((EW,), jnp.int32)]
        + [pltpu.VMEM((C,), jnp.int32)] * 4
        + [pltpu.VMEM((C, 2 * LATDIM), jnp.float32)] * 4
        + [pltpu.VMEM((C,), jnp.float32)] * 2
        + [pltpu.SemaphoreType.DMA] * 6
    ),
)


def kernel(ui_uKey, ui_iKey, uEmbeds, iEmbeds, ui_uHyper, ui_iHyper,
           rows, cols, edgeids):
    latdim = ui_uKey.shape[0] * ui_uKey.shape[2]
    uKey = jnp.transpose(ui_uKey, (1, 0, 2)).reshape(-1, latdim)
    iKey = jnp.transpose(ui_iKey, (1, 0, 2)).reshape(-1, latdim)
    U, I = _build_tables(uKey, uEmbeds, iKey, iEmbeds, ui_uHyper, ui_iHyper)
    return _edge_kernel(edgeids, rows, cols, U, I)
